# baseline (speedup 1.0000x reference)
import functools

import jax
import jax.numpy as jnp
from jax import lax
from jax.experimental import pallas as pl
from jax.experimental.pallas import tpu as pltpu
from jax.experimental.pallas import tpu_sc as plsc

BATCH = 16384
DIM = 64
ROWS = 1000000
FULL_ROWS = 999936
N_FULL_SLABS = FULL_ROWS // 128
PIECE = 4
CLAMP_SLAB = N_FULL_SLABS - PIECE
PER_W = 245
N_PIECES = 62
STAGE = 64


def _iota16():
    return lax.iota(jnp.int32, 16)


def _full16(v):
    return jnp.full((16,), v, jnp.int32)


def _gather_call(nw):
    mesh = plsc.VectorSubcoreMesh(core_axis_name="c", subcore_axis_name="s")

    @functools.partial(
        pl.kernel,
        mesh=mesh,
        out_type=jax.ShapeDtypeStruct((BATCH, 128), jnp.float32),
        scratch_types=[
            pltpu.VMEM((BATCH,), jnp.int32),
            pltpu.VMEM((BATCH,), jnp.int32),
            pltpu.VMEM((2, DIM, 128 * PIECE), jnp.float32),
            pltpu.VMEM((DIM, DIM), jnp.float32),
            pltpu.VMEM((STAGE, 128), jnp.float32),
            pltpu.VMEM((STAGE,), jnp.int32),
            pltpu.SemaphoreType.DMA,
            pltpu.SemaphoreType.DMA,
        ],
        compiler_params=pltpu.CompilerParams(
            use_tc_tiling_on_sc=True, needs_layout_passes=False
        ),
    )
    def k(x_hbm, t_hbm, tail_hbm, out_hbm,
          xv, listv, bufs, tailv, stage, oidx, sem0, sem1):
        pbuf = xv
        num_cores = lax.axis_size("c")
        wid = lax.axis_index("s") * num_cores + lax.axis_index("c")
        s0 = wid * PER_W
        n_my = jnp.minimum(PER_W, N_FULL_SLABS - s0)
        sems = (sem0, sem1)
        iota = _iota16()

        def base_slab(pp):
            return jnp.minimum(s0 + pp * PIECE, CLAMP_SLAB)

        def fetch(pp, b):
            pltpu.async_copy(
                t_hbm.at[:, pl.ds(base_slab(pp) * 128, 128 * PIECE)],
                bufs.at[b],
                sems[b],
            )

        def wait(pp, b):
            pltpu.make_async_copy(
                t_hbm.at[:, pl.ds(base_slab(pp) * 128, 128 * PIECE)],
                bufs.at[b],
                sems[b],
            ).wait()

        fetch(0, 0)
        fetch(1, 1)
        pltpu.sync_copy(x_hbm, xv)
        pltpu.sync_copy(tail_hbm, tailv)

        for q in range(STAGE // 16):
            oidx[pl.ds(q * 16, 16)] = _full16(-1)

        lim = jnp.minimum(PER_W, (ROWS + 127) // 128 - s0) * 128

        def scan_body(kk, n):
            rv = xv[pl.ds(kk * 16, 16)]
            roff = rv - s0 * 128
            m = (roff >= 0) & (roff < lim)
            cnt = plsc.all_reduce_population_count(m)[0]

            @pl.when(cnt > 0)
            def _():
                pk = ((kk * 16 + iota) << 15) | roff
                pos = n + plsc.cumsum(m.astype(jnp.int32)) - 1
                plsc.store_scatter(listv, [pos], pk, mask=m)

            return n + cnt

        n = lax.fori_loop(0, BATCH // 16, scan_body, 0)
        nv = (n + 15) // 16

        def flush():
            pltpu.sync_copy(
                stage, out_hbm.at[plsc.Indices(oidx, ignored_value=-1)]
            )
            for q in range(STAGE // 16):
                oidx[pl.ds(q * 16, 16)] = _full16(-1)

        def extract_hits(rlo, width, buf, np_, slot):
            def hbody(h, slot):
                pk16 = plsc.load_gather(pbuf, [_full16(h)])
                col = (pk16 & 32767) - rlo
                i16 = lax.shift_right_logical(pk16, 15)
                for q in range(DIM // 16):
                    vals = plsc.load_gather(buf, [iota + q * 16, col])
                    stage[slot, pl.ds(q * 16, 16)] = vals
                plsc.store_scatter(oidx, [_full16(slot)], i16, mask=iota == 0)
                s2 = slot + 1

                @pl.when(s2 == STAGE)
                def _():
                    flush()

                return jnp.where(s2 == STAGE, 0, s2)

            return lax.fori_loop(0, np_, hbody, slot)

        def compress(rlo, width):
            def cbody(v, np_):
                pk = listv[pl.ds(v * 16, 16)]
                valid = (v * 16 + iota) < n
                roff = pk & 32767
                m = valid & (roff >= rlo) & (roff < rlo + width)
                pos = np_ + plsc.cumsum(m.astype(jnp.int32)) - 1
                plsc.store_scatter(pbuf, [pos], pk, mask=m)
                return np_ + plsc.all_reduce_population_count(m)[0]

            return lax.fori_loop(0, nv, cbody, 0)

        def process_piece(pp, b, slot):
            rlo = base_slab(pp) * 128 - s0 * 128
            np_ = compress(rlo, 128 * PIECE)
            return extract_hits(rlo, 128 * PIECE, bufs.at[b], np_, slot)

        def piece_loop(g, slot):
            for b in range(2):
                pp = g * 2 + b
                wait(pp, b)
                slot = process_piece(pp, b, slot)

                @pl.when(pp + 2 < N_PIECES)
                def _(pp=pp, b=b):
                    fetch(pp + 2, b)

            return slot

        slot = lax.fori_loop(0, N_PIECES // 2, piece_loop, 0)

        tlo = FULL_ROWS - s0 * 128
        np_ = compress(tlo, DIM)
        slot = extract_hits(tlo, DIM, tailv, np_, slot)

        @pl.when(slot > 0)
        def _():
            pltpu.sync_copy(
                stage, out_hbm.at[plsc.Indices(oidx, ignored_value=-1)]
            )

    return k


def kernel(x, table):
    info = plsc.get_sparse_core_info()
    nw = info.num_cores * info.num_subcores
    table_t = table.T
    tail = table_t[:, FULL_ROWS:]
    out_padded = _gather_call(nw)(x.astype(jnp.int32), table_t, tail)
    return out_padded[:, :DIM]


# speedup vs baseline: 1.0688x; 1.0688x over previous
import functools

import jax
import jax.numpy as jnp
from jax import lax
from jax.experimental import pallas as pl
from jax.experimental.pallas import tpu as pltpu
from jax.experimental.pallas import tpu_sc as plsc

BATCH = 16384
DIM = 64
ROWS = 1000000
FULL_ROWS = 999936
N_FULL_SLABS = FULL_ROWS // 128
PIECE = 4
CLAMP_SLAB = N_FULL_SLABS - PIECE
PER_W = 245
N_PIECES = 62
STAGE = 64


def _iota16():
    return lax.iota(jnp.int32, 16)


def _full16(v):
    return jnp.full((16,), v, jnp.int32)


def _gather_call(nw):
    mesh = plsc.VectorSubcoreMesh(core_axis_name="c", subcore_axis_name="s")

    @functools.partial(
        pl.kernel,
        mesh=mesh,
        out_type=jax.ShapeDtypeStruct((BATCH, 128), jnp.float32),
        scratch_types=[
            pltpu.VMEM((BATCH,), jnp.int32),
            pltpu.VMEM((BATCH,), jnp.int32),
            pltpu.VMEM((2, DIM, 128 * PIECE), jnp.float32),
            pltpu.VMEM((DIM, DIM), jnp.float32),
            pltpu.VMEM((STAGE, 128), jnp.float32),
            pltpu.VMEM((STAGE,), jnp.int32),
            pltpu.SemaphoreType.DMA,
            pltpu.SemaphoreType.DMA,
        ],
        compiler_params=pltpu.CompilerParams(
            use_tc_tiling_on_sc=True, needs_layout_passes=False
        ),
    )
    def k(x_hbm, t_hbm, tail_hbm, out_hbm,
          xv, listv, bufs, tailv, stage, oidx, sem0, sem1):
        pbuf = xv
        num_cores = lax.axis_size("c")
        wid = lax.axis_index("s") * num_cores + lax.axis_index("c")
        s0 = wid * PER_W
        n_my = jnp.minimum(PER_W, N_FULL_SLABS - s0)
        sems = (sem0, sem1)
        iota = _iota16()

        def base_slab(pp):
            return jnp.minimum(s0 + pp * PIECE, CLAMP_SLAB)

        def fetch(pp, b):
            pltpu.async_copy(
                t_hbm.at[:, pl.ds(base_slab(pp) * 128, 128 * PIECE)],
                bufs.at[b],
                sems[b],
            )

        def wait(pp, b):
            pltpu.make_async_copy(
                t_hbm.at[:, pl.ds(base_slab(pp) * 128, 128 * PIECE)],
                bufs.at[b],
                sems[b],
            ).wait()

        fetch(0, 0)
        fetch(1, 1)
        pltpu.sync_copy(x_hbm, xv)
        pltpu.sync_copy(tail_hbm, tailv)

        for q in range(STAGE // 16):
            oidx[pl.ds(q * 16, 16)] = _full16(-1)

        lim = jnp.minimum(PER_W, (ROWS + 127) // 128 - s0) * 128

        def scan_body(kk, n):
            rv = xv[pl.ds(kk * 16, 16)]
            roff = rv - s0 * 128
            m = (roff >= 0) & (roff < lim)
            pk = ((kk * 16 + iota) << 15) | roff
            pos = n + plsc.cumsum(m.astype(jnp.int32)) - 1
            plsc.store_scatter(listv, [pos], pk, mask=m)
            return n + plsc.all_reduce_population_count(m)[0]

        n = lax.fori_loop(0, BATCH // 16, scan_body, 0)
        nv = (n + 15) // 16

        def flush():
            pltpu.sync_copy(
                stage, out_hbm.at[plsc.Indices(oidx, ignored_value=-1)]
            )
            for q in range(STAGE // 16):
                oidx[pl.ds(q * 16, 16)] = _full16(-1)

        def extract_hits(rlo, width, buf, np_, slot):
            def hbody(h, slot):
                pk16 = plsc.load_gather(pbuf, [_full16(h)])
                col = (pk16 & 32767) - rlo
                i16 = lax.shift_right_logical(pk16, 15)
                for q in range(DIM // 16):
                    vals = plsc.load_gather(buf, [iota + q * 16, col])
                    stage[slot, pl.ds(q * 16, 16)] = vals
                plsc.store_scatter(oidx, [_full16(slot)], i16, mask=iota == 0)
                s2 = slot + 1

                @pl.when(s2 == STAGE)
                def _():
                    flush()

                return jnp.where(s2 == STAGE, 0, s2)

            return lax.fori_loop(0, np_, hbody, slot)

        def compress(rlo, width):
            def cbody(v, np_):
                pk = listv[pl.ds(v * 16, 16)]
                valid = (v * 16 + iota) < n
                roff = pk & 32767
                m = valid & (roff >= rlo) & (roff < rlo + width)
                pos = np_ + plsc.cumsum(m.astype(jnp.int32)) - 1
                plsc.store_scatter(pbuf, [pos], pk, mask=m)
                return np_ + plsc.all_reduce_population_count(m)[0]

            return lax.fori_loop(0, nv, cbody, 0)

        def process_piece(pp, b, slot):
            rlo = base_slab(pp) * 128 - s0 * 128
            np_ = compress(rlo, 128 * PIECE)
            return extract_hits(rlo, 128 * PIECE, bufs.at[b], np_, slot)

        def piece_loop(g, slot):
            for b in range(2):
                pp = g * 2 + b
                wait(pp, b)
                slot = process_piece(pp, b, slot)

                @pl.when(pp + 2 < N_PIECES)
                def _(pp=pp, b=b):
                    fetch(pp + 2, b)

            return slot

        slot = lax.fori_loop(0, N_PIECES // 2, piece_loop, 0)

        tlo = FULL_ROWS - s0 * 128
        np_ = compress(tlo, DIM)
        slot = extract_hits(tlo, DIM, tailv, np_, slot)

        @pl.when(slot > 0)
        def _():
            pltpu.sync_copy(
                stage, out_hbm.at[plsc.Indices(oidx, ignored_value=-1)]
            )

    return k


def kernel(x, table):
    info = plsc.get_sparse_core_info()
    nw = info.num_cores * info.num_subcores
    table_t = table.T
    tail = table_t[:, FULL_ROWS:]
    out_padded = _gather_call(nw)(x.astype(jnp.int32), table_t, tail)
    return out_padded[:, :DIM]


# speedup vs baseline: 1.0705x; 1.0016x over previous
import functools

import jax
import jax.numpy as jnp
from jax import lax
from jax.experimental import pallas as pl
from jax.experimental.pallas import tpu as pltpu
from jax.experimental.pallas import tpu_sc as plsc

BATCH = 16384
DIM = 64
ROWS = 1000000
FULL_ROWS = 999936
N_FULL_SLABS = FULL_ROWS // 128
PIECE = 4
CLAMP_SLAB = N_FULL_SLABS - PIECE
PER_W = 245
N_PIECES = 62
STAGE = 64


def _iota16():
    return lax.iota(jnp.int32, 16)


def _full16(v):
    return jnp.full((16,), v, jnp.int32)


def _gather_call(nw):
    mesh = plsc.VectorSubcoreMesh(core_axis_name="c", subcore_axis_name="s")

    @functools.partial(
        pl.kernel,
        mesh=mesh,
        out_type=jax.ShapeDtypeStruct((BATCH, 128), jnp.float32),
        scratch_types=[
            pltpu.VMEM((BATCH,), jnp.int32),
            pltpu.VMEM((BATCH,), jnp.int32),
            pltpu.VMEM((2, DIM, 128 * PIECE), jnp.float32),
            pltpu.VMEM((DIM, DIM), jnp.float32),
            pltpu.VMEM((STAGE, 128), jnp.float32),
            pltpu.VMEM((STAGE,), jnp.int32),
            pltpu.SemaphoreType.DMA,
            pltpu.SemaphoreType.DMA,
        ],
        compiler_params=pltpu.CompilerParams(
            use_tc_tiling_on_sc=True, needs_layout_passes=False
        ),
    )
    def k(x_hbm, t_hbm, tail_hbm, out_hbm,
          xv, listv, bufs, tailv, stage, oidx, sem0, sem1):
        pbuf = xv
        num_cores = lax.axis_size("c")
        wid = lax.axis_index("s") * num_cores + lax.axis_index("c")
        s0 = wid * PER_W
        n_my = jnp.minimum(PER_W, N_FULL_SLABS - s0)
        sems = (sem0, sem1)
        iota = _iota16()

        def base_slab(pp):
            return jnp.minimum(s0 + pp * PIECE, CLAMP_SLAB)

        def fetch(pp, b):
            pltpu.async_copy(
                t_hbm.at[:, pl.ds(base_slab(pp) * 128, 128 * PIECE)],
                bufs.at[b],
                sems[b],
            )

        def wait(pp, b):
            pltpu.make_async_copy(
                t_hbm.at[:, pl.ds(base_slab(pp) * 128, 128 * PIECE)],
                bufs.at[b],
                sems[b],
            ).wait()

        pltpu.sync_copy(x_hbm, xv)
        fetch(0, 0)
        fetch(1, 1)
        pltpu.sync_copy(tail_hbm, tailv)

        for q in range(STAGE // 16):
            oidx[pl.ds(q * 16, 16)] = _full16(-1)

        lim = jnp.minimum(PER_W, (ROWS + 127) // 128 - s0) * 128

        def scan_body(kk, n):
            rv = xv[pl.ds(kk * 16, 16)]
            roff = rv - s0 * 128
            m = (roff >= 0) & (roff < lim)
            pk = ((kk * 16 + iota) << 15) | roff
            pos = n + plsc.cumsum(m.astype(jnp.int32)) - 1
            plsc.store_scatter(listv, [pos], pk, mask=m)
            return n + plsc.all_reduce_population_count(m)[0]

        n = lax.fori_loop(0, BATCH // 16, scan_body, 0)
        nv = (n + 15) // 16

        def flush():
            pltpu.sync_copy(
                stage, out_hbm.at[plsc.Indices(oidx, ignored_value=-1)]
            )
            for q in range(STAGE // 16):
                oidx[pl.ds(q * 16, 16)] = _full16(-1)

        def extract_hits(rlo, width, buf, np_, slot):
            def hbody(h, slot):
                pk16 = plsc.load_gather(pbuf, [_full16(h)])
                col = (pk16 & 32767) - rlo
                i16 = lax.shift_right_logical(pk16, 15)
                for q in range(DIM // 16):
                    vals = plsc.load_gather(buf, [iota + q * 16, col])
                    stage[slot, pl.ds(q * 16, 16)] = vals
                plsc.store_scatter(oidx, [_full16(slot)], i16, mask=iota == 0)
                s2 = slot + 1

                @pl.when(s2 == STAGE)
                def _():
                    flush()

                return jnp.where(s2 == STAGE, 0, s2)

            return lax.fori_loop(0, np_, hbody, slot)

        def compress(rlo, width):
            def cbody(v, np_):
                pk = listv[pl.ds(v * 16, 16)]
                valid = (v * 16 + iota) < n
                roff = pk & 32767
                m = valid & (roff >= rlo) & (roff < rlo + width)
                pos = np_ + plsc.cumsum(m.astype(jnp.int32)) - 1
                plsc.store_scatter(pbuf, [pos], pk, mask=m)
                return np_ + plsc.all_reduce_population_count(m)[0]

            return lax.fori_loop(0, nv, cbody, 0)

        def process_piece(pp, b, slot):
            rlo = base_slab(pp) * 128 - s0 * 128
            np_ = compress(rlo, 128 * PIECE)
            return extract_hits(rlo, 128 * PIECE, bufs.at[b], np_, slot)

        def piece_loop(g, slot):
            for b in range(2):
                pp = g * 2 + b
                wait(pp, b)
                slot = process_piece(pp, b, slot)

                @pl.when(pp + 2 < N_PIECES)
                def _(pp=pp, b=b):
                    fetch(pp + 2, b)

            return slot

        slot = lax.fori_loop(0, N_PIECES // 2, piece_loop, 0)

        tlo = FULL_ROWS - s0 * 128
        np_ = compress(tlo, DIM)
        slot = extract_hits(tlo, DIM, tailv, np_, slot)

        @pl.when(slot > 0)
        def _():
            pltpu.sync_copy(
                stage, out_hbm.at[plsc.Indices(oidx, ignored_value=-1)]
            )

    return k


def kernel(x, table):
    info = plsc.get_sparse_core_info()
    nw = info.num_cores * info.num_subcores
    table_t = table.T
    tail = table_t[:, FULL_ROWS:]
    out_padded = _gather_call(nw)(x.astype(jnp.int32), table_t, tail)
    return out_padded[:, :DIM]


# speedup vs baseline: 1.0951x; 1.0230x over previous
import functools

import jax
import jax.numpy as jnp
from jax import lax
from jax.experimental import pallas as pl
from jax.experimental.pallas import tpu as pltpu
from jax.experimental.pallas import tpu_sc as plsc

BATCH = 16384
DIM = 64
ROWS = 1000000
FULL_ROWS = 999936
N_FULL_SLABS = FULL_ROWS // 128
PIECE = 4
CLAMP_SLAB = N_FULL_SLABS - PIECE
PER_W = 245
N_PIECES = 62
STAGE = 64


def _iota16():
    return lax.iota(jnp.int32, 16)


def _full16(v):
    return jnp.full((16,), v, jnp.int32)


def _gather_call(nw):
    mesh = plsc.VectorSubcoreMesh(core_axis_name="c", subcore_axis_name="s")

    @functools.partial(
        pl.kernel,
        mesh=mesh,
        out_type=jax.ShapeDtypeStruct((BATCH, 128), jnp.float32),
        scratch_types=[
            pltpu.VMEM((BATCH,), jnp.int32),
            pltpu.VMEM((BATCH,), jnp.int32),
            pltpu.VMEM((2, DIM, 128 * PIECE), jnp.float32),
            pltpu.VMEM((DIM, DIM), jnp.float32),
            pltpu.VMEM((STAGE, 128), jnp.float32),
            pltpu.VMEM((STAGE,), jnp.int32),
            pltpu.SemaphoreType.DMA,
            pltpu.SemaphoreType.DMA,
            pltpu.SemaphoreType.DMA,
        ],
        compiler_params=pltpu.CompilerParams(
            use_tc_tiling_on_sc=True, needs_layout_passes=False
        ),
    )
    def k(x_hbm, t_hbm, tail_hbm, out_hbm,
          xv, listv, bufs, tailv, stage, oidx, sem0, sem1, tsem):
        pbuf = xv
        num_cores = lax.axis_size("c")
        wid = lax.axis_index("s") * num_cores + lax.axis_index("c")
        s0 = wid * PER_W
        n_my = jnp.minimum(PER_W, N_FULL_SLABS - s0)
        sems = (sem0, sem1)
        iota = _iota16()

        def base_slab(pp):
            return jnp.minimum(s0 + pp * PIECE, CLAMP_SLAB)

        def fetch(pp, b):
            pltpu.async_copy(
                t_hbm.at[:, pl.ds(base_slab(pp) * 128, 128 * PIECE)],
                bufs.at[b],
                sems[b],
            )

        def wait(pp, b):
            pltpu.make_async_copy(
                t_hbm.at[:, pl.ds(base_slab(pp) * 128, 128 * PIECE)],
                bufs.at[b],
                sems[b],
            ).wait()

        pltpu.sync_copy(x_hbm, xv)
        fetch(0, 0)
        fetch(1, 1)
        pltpu.async_copy(tail_hbm, tailv, tsem)

        for q in range(STAGE // 16):
            oidx[pl.ds(q * 16, 16)] = _full16(-1)

        lim = jnp.minimum(PER_W, (ROWS + 127) // 128 - s0) * 128

        def scan_body(kk, n):
            rv = xv[pl.ds(kk * 16, 16)]
            roff = rv - s0 * 128
            m = (roff >= 0) & (roff < lim)
            pk = ((kk * 16 + iota) << 15) | roff
            pos = n + plsc.cumsum(m.astype(jnp.int32)) - 1
            plsc.store_scatter(listv, [pos], pk, mask=m)
            return n + plsc.all_reduce_population_count(m)[0]

        n = lax.fori_loop(0, BATCH // 16, scan_body, 0)
        nv = (n + 15) // 16

        def flush():
            pltpu.sync_copy(
                stage, out_hbm.at[plsc.Indices(oidx, ignored_value=-1)]
            )
            for q in range(STAGE // 16):
                oidx[pl.ds(q * 16, 16)] = _full16(-1)

        def extract_hits(rlo, width, buf, np_, slot):
            def hbody(h, slot):
                pk16 = plsc.load_gather(pbuf, [_full16(h)])
                col = (pk16 & 32767) - rlo
                i16 = lax.shift_right_logical(pk16, 15)
                for q in range(DIM // 16):
                    vals = plsc.load_gather(buf, [iota + q * 16, col])
                    stage[slot, pl.ds(q * 16, 16)] = vals
                plsc.store_scatter(oidx, [_full16(slot)], i16, mask=iota == 0)
                s2 = slot + 1

                @pl.when(s2 == STAGE)
                def _():
                    flush()

                return jnp.where(s2 == STAGE, 0, s2)

            return lax.fori_loop(0, np_, hbody, slot)

        def compress(rlo, width):
            def cbody(v, np_):
                pk = listv[pl.ds(v * 16, 16)]
                valid = (v * 16 + iota) < n
                roff = pk & 32767
                m = valid & (roff >= rlo) & (roff < rlo + width)
                pos = np_ + plsc.cumsum(m.astype(jnp.int32)) - 1
                plsc.store_scatter(pbuf, [pos], pk, mask=m)
                return np_ + plsc.all_reduce_population_count(m)[0]

            return lax.fori_loop(0, nv, cbody, 0)

        def process_piece(pp, b, slot):
            rlo = base_slab(pp) * 128 - s0 * 128
            np_ = compress(rlo, 128 * PIECE)
            return extract_hits(rlo, 128 * PIECE, bufs.at[b], np_, slot)

        def piece_loop(g, slot):
            for b in range(2):
                pp = g * 2 + b
                wait(pp, b)
                slot = process_piece(pp, b, slot)

                @pl.when(pp + 2 < N_PIECES)
                def _(pp=pp, b=b):
                    fetch(pp + 2, b)

            return slot

        slot = lax.fori_loop(0, N_PIECES // 2, piece_loop, 0)

        pltpu.make_async_copy(tail_hbm, tailv, tsem).wait()
        tlo = FULL_ROWS - s0 * 128
        np_ = compress(tlo, DIM)
        slot = extract_hits(tlo, DIM, tailv, np_, slot)

        @pl.when(slot > 0)
        def _():
            pltpu.sync_copy(
                stage, out_hbm.at[plsc.Indices(oidx, ignored_value=-1)]
            )

    return k


def kernel(x, table):
    info = plsc.get_sparse_core_info()
    nw = info.num_cores * info.num_subcores
    table_t = table.T
    tail = table_t[:, FULL_ROWS:]
    out_padded = _gather_call(nw)(x.astype(jnp.int32), table_t, tail)
    return out_padded[:, :DIM]


# speedup vs baseline: 1.1104x; 1.0140x over previous
import functools

import jax
import jax.numpy as jnp
from jax import lax
from jax.experimental import pallas as pl
from jax.experimental.pallas import tpu as pltpu
from jax.experimental.pallas import tpu_sc as plsc

BATCH = 16384
DIM = 64
ROWS = 1000000
FULL_ROWS = 999936
N_FULL_SLABS = FULL_ROWS // 128
PIECE = 4
CLAMP_SLAB = N_FULL_SLABS - PIECE
PER_W = 245
N_PIECES = 62
STAGE = 64


def _iota16():
    return lax.iota(jnp.int32, 16)


def _full16(v):
    return jnp.full((16,), v, jnp.int32)


def _gather_call(nw):
    mesh = plsc.VectorSubcoreMesh(core_axis_name="c", subcore_axis_name="s")

    @functools.partial(
        pl.kernel,
        mesh=mesh,
        out_type=jax.ShapeDtypeStruct((BATCH, 128), jnp.float32),
        scratch_types=[
            pltpu.VMEM((BATCH,), jnp.int32),
            pltpu.VMEM((BATCH,), jnp.int32),
            pltpu.VMEM((2, DIM, 128 * PIECE), jnp.float32),
            pltpu.VMEM((DIM, DIM), jnp.float32),
            pltpu.VMEM((STAGE, 128), jnp.float32),
            pltpu.VMEM((STAGE,), jnp.int32),
            pltpu.SemaphoreType.DMA,
            pltpu.SemaphoreType.DMA,
            pltpu.SemaphoreType.DMA,
        ],
        compiler_params=pltpu.CompilerParams(
            use_tc_tiling_on_sc=True, needs_layout_passes=False
        ),
    )
    def k(x_hbm, t_hbm, tail_hbm, out_hbm,
          xv, listv, bufs, tailv, stage, oidx, sem0, sem1, tsem):
        pbuf = xv
        num_cores = lax.axis_size("c")
        wid = lax.axis_index("s") * num_cores + lax.axis_index("c")
        s0 = wid * PER_W
        n_my = jnp.minimum(PER_W, N_FULL_SLABS - s0)
        sems = (sem0, sem1)
        iota = _iota16()

        def base_slab(pp):
            return jnp.minimum(s0 + pp * PIECE, CLAMP_SLAB)

        def fetch(pp, b):
            pltpu.async_copy(
                t_hbm.at[:, pl.ds(base_slab(pp) * 128, 128 * PIECE)],
                bufs.at[b],
                sems[b],
            )

        def wait(pp, b):
            pltpu.make_async_copy(
                t_hbm.at[:, pl.ds(base_slab(pp) * 128, 128 * PIECE)],
                bufs.at[b],
                sems[b],
            ).wait()

        pltpu.sync_copy(x_hbm, xv)
        fetch(0, 0)
        fetch(1, 1)
        pltpu.async_copy(tail_hbm, tailv, tsem)

        for q in range(STAGE // 16):
            oidx[pl.ds(q * 16, 16)] = _full16(-1)

        lim = jnp.minimum(PER_W, (ROWS + 127) // 128 - s0) * 128

        def scan_body(kk2, n):
            rv0 = xv[pl.ds(kk2 * 32, 16)]
            rv1 = xv[pl.ds(kk2 * 32 + 16, 16)]
            roff0 = rv0 - s0 * 128
            roff1 = rv1 - s0 * 128
            m0 = (roff0 >= 0) & (roff0 < lim)
            m1 = (roff1 >= 0) & (roff1 < lim)
            cs0 = plsc.cumsum(m0.astype(jnp.int32))
            cs1 = plsc.cumsum(m1.astype(jnp.int32))
            pk0 = ((kk2 * 32 + iota) << 15) | roff0
            pk1 = ((kk2 * 32 + 16 + iota) << 15) | roff1
            c0 = cs0[15]
            plsc.store_scatter(listv, [n + cs0 - 1], pk0, mask=m0)
            plsc.store_scatter(listv, [n + c0 + cs1 - 1], pk1, mask=m1)
            return n + c0 + cs1[15]

        n = lax.fori_loop(0, BATCH // 32, scan_body, 0)
        nv = (n + 15) // 16

        def flush():
            pltpu.sync_copy(
                stage, out_hbm.at[plsc.Indices(oidx, ignored_value=-1)]
            )
            for q in range(STAGE // 16):
                oidx[pl.ds(q * 16, 16)] = _full16(-1)

        def extract_hits(rlo, width, buf, np_, slot):
            def hbody(h, slot):
                pk16 = plsc.load_gather(pbuf, [_full16(h)])
                col = (pk16 & 32767) - rlo
                i16 = lax.shift_right_logical(pk16, 15)
                for q in range(DIM // 16):
                    vals = plsc.load_gather(buf, [iota + q * 16, col])
                    stage[slot, pl.ds(q * 16, 16)] = vals
                plsc.store_scatter(oidx, [_full16(slot)], i16, mask=iota == 0)
                s2 = slot + 1

                @pl.when(s2 == STAGE)
                def _():
                    flush()

                return jnp.where(s2 == STAGE, 0, s2)

            return lax.fori_loop(0, np_, hbody, slot)

        def compress(rlo, width):
            def cbody(v, np_):
                pk = listv[pl.ds(v * 16, 16)]
                valid = (v * 16 + iota) < n
                roff = pk & 32767
                m = valid & (roff >= rlo) & (roff < rlo + width)
                cs = plsc.cumsum(m.astype(jnp.int32))
                plsc.store_scatter(pbuf, [np_ + cs - 1], pk, mask=m)
                return np_ + cs[15]

            return lax.fori_loop(0, nv, cbody, 0)

        def process_piece(pp, b, slot):
            rlo = base_slab(pp) * 128 - s0 * 128
            np_ = compress(rlo, 128 * PIECE)
            return extract_hits(rlo, 128 * PIECE, bufs.at[b], np_, slot)

        def piece_loop(g, slot):
            for b in range(2):
                pp = g * 2 + b
                wait(pp, b)
                slot = process_piece(pp, b, slot)

                @pl.when(pp + 2 < N_PIECES)
                def _(pp=pp, b=b):
                    fetch(pp + 2, b)

            return slot

        slot = lax.fori_loop(0, N_PIECES // 2, piece_loop, 0)

        pltpu.make_async_copy(tail_hbm, tailv, tsem).wait()
        tlo = FULL_ROWS - s0 * 128
        np_ = compress(tlo, DIM)
        slot = extract_hits(tlo, DIM, tailv, np_, slot)

        @pl.when(slot > 0)
        def _():
            pltpu.sync_copy(
                stage, out_hbm.at[plsc.Indices(oidx, ignored_value=-1)]
            )

    return k


def kernel(x, table):
    info = plsc.get_sparse_core_info()
    nw = info.num_cores * info.num_subcores
    table_t = table.T
    tail = table_t[:, FULL_ROWS:]
    out_padded = _gather_call(nw)(x.astype(jnp.int32), table_t, tail)
    return out_padded[:, :DIM]


# speedup vs baseline: 1.1128x; 1.0021x over previous
import functools

import jax
import jax.numpy as jnp
from jax import lax
from jax.experimental import pallas as pl
from jax.experimental.pallas import tpu as pltpu
from jax.experimental.pallas import tpu_sc as plsc

BATCH = 16384
DIM = 64
ROWS = 1000000
FULL_ROWS = 999936
N_FULL_SLABS = FULL_ROWS // 128
PIECE = 4
CLAMP_SLAB = N_FULL_SLABS - PIECE
PER_W = 245
N_PIECES = 62
STAGE = 64


def _iota16():
    return lax.iota(jnp.int32, 16)


def _full16(v):
    return jnp.full((16,), v, jnp.int32)


def _gather_call(nw):
    mesh = plsc.VectorSubcoreMesh(core_axis_name="c", subcore_axis_name="s")

    @functools.partial(
        pl.kernel,
        mesh=mesh,
        out_type=jax.ShapeDtypeStruct((BATCH, 128), jnp.float32),
        scratch_types=[
            pltpu.VMEM((BATCH,), jnp.int32),
            pltpu.VMEM((BATCH,), jnp.int32),
            pltpu.VMEM((2, DIM, 128 * PIECE), jnp.float32),
            pltpu.VMEM((DIM, DIM), jnp.float32),
            pltpu.VMEM((2 * STAGE, 128), jnp.float32),
            pltpu.VMEM((2 * STAGE,), jnp.int32),
            pltpu.SemaphoreType.DMA,
            pltpu.SemaphoreType.DMA,
            pltpu.SemaphoreType.DMA,
            pltpu.SemaphoreType.DMA,
        ],
        compiler_params=pltpu.CompilerParams(
            use_tc_tiling_on_sc=True, needs_layout_passes=False
        ),
    )
    def k(x_hbm, t_hbm, tail_hbm, out_hbm,
          xv, listv, bufs, tailv, stage, oidx, sem0, sem1, tsem, fsem):
        pbuf = xv
        num_cores = lax.axis_size("c")
        wid = lax.axis_index("s") * num_cores + lax.axis_index("c")
        s0 = wid * PER_W
        n_my = jnp.minimum(PER_W, N_FULL_SLABS - s0)
        sems = (sem0, sem1)
        iota = _iota16()

        def base_slab(pp):
            return jnp.minimum(s0 + pp * PIECE, CLAMP_SLAB)

        def fetch(pp, b):
            pltpu.async_copy(
                t_hbm.at[:, pl.ds(base_slab(pp) * 128, 128 * PIECE)],
                bufs.at[b],
                sems[b],
            )

        def wait(pp, b):
            pltpu.make_async_copy(
                t_hbm.at[:, pl.ds(base_slab(pp) * 128, 128 * PIECE)],
                bufs.at[b],
                sems[b],
            ).wait()

        pltpu.sync_copy(x_hbm, xv)
        fetch(0, 0)
        fetch(1, 1)
        pltpu.async_copy(tail_hbm, tailv, tsem)

        for q in range(2 * STAGE // 16):
            oidx[pl.ds(q * 16, 16)] = _full16(-1)

        lim = jnp.minimum(PER_W, (ROWS + 127) // 128 - s0) * 128

        def scan_body(kk2, n):
            rv0 = xv[pl.ds(kk2 * 32, 16)]
            rv1 = xv[pl.ds(kk2 * 32 + 16, 16)]
            roff0 = rv0 - s0 * 128
            roff1 = rv1 - s0 * 128
            m0 = (roff0 >= 0) & (roff0 < lim)
            m1 = (roff1 >= 0) & (roff1 < lim)
            cs0 = plsc.cumsum(m0.astype(jnp.int32))
            cs1 = plsc.cumsum(m1.astype(jnp.int32))
            pk0 = ((kk2 * 32 + iota) << 15) | roff0
            pk1 = ((kk2 * 32 + 16 + iota) << 15) | roff1
            c0 = cs0[15]
            plsc.store_scatter(listv, [n + cs0 - 1], pk0, mask=m0)
            plsc.store_scatter(listv, [n + c0 + cs1 - 1], pk1, mask=m1)
            return n + c0 + cs1[15]

        n = lax.fori_loop(0, BATCH // 32, scan_body, 0)
        nv = (n + 15) // 16

        def flush_refs(f):
            return (
                stage.at[pl.ds(f * STAGE, STAGE)],
                out_hbm.at[
                    plsc.Indices(
                        oidx.at[pl.ds(f * STAGE, STAGE)], ignored_value=-1
                    )
                ],
            )

        def start_flush(f):
            src, dst = flush_refs(f)
            pltpu.async_copy(src, dst, fsem)

        def wait_flush(f):
            src, dst = flush_refs(f)
            pltpu.make_async_copy(src, dst, fsem).wait()

        def extract_hits(rlo, width, buf, np_, carry_in):
            def hbody(h, carry):
                slot, flushed = carry
                pk16 = plsc.load_gather(pbuf, [_full16(h)])
                col = (pk16 & 32767) - rlo
                i16 = lax.shift_right_logical(pk16, 15)
                for q in range(DIM // 16):
                    vals = plsc.load_gather(buf, [iota + q * 16, col])
                    stage[slot, pl.ds(q * 16, 16)] = vals
                plsc.store_scatter(oidx, [_full16(slot)], i16, mask=iota == 0)
                s2 = slot + 1

                @pl.when(s2 % STAGE == 0)
                def _():
                    filled = (s2 // STAGE - 1) % 2
                    other = 1 - filled
                    start_flush(filled)

                    @pl.when(flushed != 0)
                    def _():
                        wait_flush(other)

                    for q in range(STAGE // 16):
                        oidx[pl.ds(other * STAGE + q * 16, 16)] = _full16(-1)

                return (
                    jnp.where(s2 == 2 * STAGE, 0, s2),
                    jnp.where(s2 % STAGE == 0, 1, flushed),
                )

            return lax.fori_loop(0, np_, hbody, carry_in)

        def compress(rlo, width):
            def cbody(v, np_):
                pk = listv[pl.ds(v * 16, 16)]
                valid = (v * 16 + iota) < n
                roff = pk & 32767
                m = valid & (roff >= rlo) & (roff < rlo + width)
                cs = plsc.cumsum(m.astype(jnp.int32))
                plsc.store_scatter(pbuf, [np_ + cs - 1], pk, mask=m)
                return np_ + cs[15]

            return lax.fori_loop(0, nv, cbody, 0)

        def process_piece(pp, b, carry):
            rlo = base_slab(pp) * 128 - s0 * 128
            np_ = compress(rlo, 128 * PIECE)
            return extract_hits(rlo, 128 * PIECE, bufs.at[b], np_, carry)

        def piece_loop(g, carry):
            for b in range(2):
                pp = g * 2 + b
                wait(pp, b)
                carry = process_piece(pp, b, carry)

                @pl.when(pp + 2 < N_PIECES)
                def _(pp=pp, b=b):
                    fetch(pp + 2, b)

            return carry

        carry = lax.fori_loop(0, N_PIECES // 2, piece_loop, (0, 0))

        pltpu.make_async_copy(tail_hbm, tailv, tsem).wait()
        tlo = FULL_ROWS - s0 * 128
        np_ = compress(tlo, DIM)
        slot, flushed = extract_hits(tlo, DIM, tailv, np_, carry)

        cur = slot // STAGE

        @pl.when(flushed != 0)
        def _():
            wait_flush(1 - cur)

        @pl.when(slot % STAGE != 0)
        def _():
            start_flush(cur)
            wait_flush(cur)

    return k


def kernel(x, table):
    info = plsc.get_sparse_core_info()
    nw = info.num_cores * info.num_subcores
    table_t = table.T
    tail = table_t[:, FULL_ROWS:]
    out_padded = _gather_call(nw)(x.astype(jnp.int32), table_t, tail)
    return out_padded[:, :DIM]
